# baseline (speedup 1.0000x reference)
import jax
import jax.numpy as jnp
from jax.experimental import pallas as pl
from jax.experimental.pallas import tpu as pltpu

B, INPUT_DIM, HIDDEN_DIM, EMBED_DIM, K = 2048, 768, 2048, 256, 8192
BB = 256
NCHAIN = 4
COMMIT = 1.25


def _dot_t(a, b):
    return jax.lax.dot_general(a, b, (((1,), (1,)), ((), ())),
                               preferred_element_type=jnp.float32)


def _body(x_ref, t_ref, w1_ref, b1_ref, w2_ref, b2_ref,
          dw1_ref, db1_ref, dw2_ref, db2_ref,
          cb0_ref, cb1_ref, cb2_ref, out_ref, sq_ref):
    inv_t = 1.0 / t_ref[0]

    @pl.when(pl.program_id(0) == 0)
    def _init():
        ones = jnp.ones((1, EMBED_DIM), jnp.float32)
        for i, cb_ref in enumerate((cb0_ref, cb1_ref, cb2_ref)):
            cb = cb_ref[...]
            sq_ref[i:i + 1, :] = _dot_t(ones, cb * cb) * inv_t
        out_ref[...] = jnp.zeros((1, 1), jnp.float32)

    x = x_ref[...]
    h = jnp.maximum(
        jnp.dot(x, w1_ref[...], preferred_element_type=jnp.float32)
        + b1_ref[...], 0.0)
    res = jnp.dot(h, w2_ref[...], preferred_element_type=jnp.float32) + b2_ref[...]

    CH = BB // NCHAIN
    r = [res[j * CH:(j + 1) * CH] for j in range(NCHAIN)]
    q = [jnp.zeros((CH, EMBED_DIM), jnp.float32) for _ in range(NCHAIN)]
    rqs = [jnp.zeros((CH, 1), jnp.float32) for _ in range(NCHAIN)]
    two_inv_t = 2.0 * inv_t
    for i, cb_ref in enumerate((cb0_ref, cb1_ref, cb2_ref)):
        cb = cb_ref[...]
        sq = sq_ref[i:i + 1, :]
        lg = [_dot_t(r[j] * two_inv_t, cb) - sq for j in range(NCHAIN)]
        for j in range(NCHAIN):
            m = jnp.max(lg[j], axis=1, keepdims=True)
            e = jnp.exp(lg[j] - m)
            d = jnp.sum(e, axis=1, keepdims=True)
            emb = jnp.dot(e, cb, preferred_element_type=jnp.float32) / d
            r[j] = r[j] - emb
            q[j] = q[j] + emb
            rqs[j] = rqs[j] + COMMIT * jnp.sum(r[j] * r[j], axis=1,
                                               keepdims=True)

    quant = jnp.concatenate(q, axis=0)
    rq = jnp.concatenate(rqs, axis=0)
    hd = jnp.maximum(
        jnp.dot(quant, dw1_ref[...], preferred_element_type=jnp.float32)
        + db1_ref[...], 0.0)
    x_hat = jnp.dot(hd, dw2_ref[...], preferred_element_type=jnp.float32) + db2_ref[...]
    diff = x_hat - x
    recon = jnp.sum(diff * diff, axis=1, keepdims=True)
    out_ref[...] += jnp.sum(recon + rq).reshape(1, 1) / B


def kernel(x, gumbel_t, enc_W1, enc_b1, enc_W2, enc_b2,
           dec_W1, dec_b1, dec_W2, dec_b2, cb0, cb1, cb2):
    t = jnp.asarray(gumbel_t, jnp.float32).reshape(1)
    b1 = enc_b1.reshape(1, HIDDEN_DIM)
    b2 = enc_b2.reshape(1, EMBED_DIM)
    db1 = dec_b1.reshape(1, HIDDEN_DIM)
    db2 = dec_b2.reshape(1, INPUT_DIM)

    const = lambda i: (0, 0)
    out = pl.pallas_call(
        _body,
        grid=(B // BB,),
        in_specs=[
            pl.BlockSpec((BB, INPUT_DIM), lambda i: (i, 0)),
            pl.BlockSpec(memory_space=pltpu.SMEM),
            pl.BlockSpec((INPUT_DIM, HIDDEN_DIM), const),
            pl.BlockSpec((1, HIDDEN_DIM), const),
            pl.BlockSpec((HIDDEN_DIM, EMBED_DIM), const),
            pl.BlockSpec((1, EMBED_DIM), const),
            pl.BlockSpec((EMBED_DIM, HIDDEN_DIM), const),
            pl.BlockSpec((1, HIDDEN_DIM), const),
            pl.BlockSpec((HIDDEN_DIM, INPUT_DIM), const),
            pl.BlockSpec((1, INPUT_DIM), const),
            pl.BlockSpec((K, EMBED_DIM), const),
            pl.BlockSpec((K, EMBED_DIM), const),
            pl.BlockSpec((K, EMBED_DIM), const),
        ],
        out_specs=pl.BlockSpec((1, 1), const),
        out_shape=jax.ShapeDtypeStruct((1, 1), jnp.float32),
        scratch_shapes=[pltpu.VMEM((8, K), jnp.float32)],
        compiler_params=pltpu.CompilerParams(
            dimension_semantics=("arbitrary",)),
    )(x, t, enc_W1, b1, enc_W2, b2, dec_W1, db1, dec_W2, db2, cb0, cb1, cb2)
    return out[0, 0]


# speedup vs baseline: 1.7887x; 1.7887x over previous
import jax
import jax.numpy as jnp
from jax.experimental import pallas as pl
from jax.experimental.pallas import tpu as pltpu

B, INPUT_DIM, HIDDEN_DIM, EMBED_DIM, K = 2048, 768, 2048, 256, 8192
BB = 512
NCHAIN = 2
COMMIT = 1.25


def _dot_t(a, b):
    return jax.lax.dot_general(a, b, (((1,), (1,)), ((), ())),
                               preferred_element_type=jnp.float32)


def _body(x_ref, t_ref, w1_ref, b1_ref, w2_ref, b2_ref,
          dw1_ref, db1_ref, dw2_ref, db2_ref,
          cb0_ref, cb1_ref, cb2_ref, out_ref, sq_ref):
    inv_t = 1.0 / t_ref[0]

    @pl.when(pl.program_id(0) == 0)
    def _init():
        ones = jnp.ones((1, EMBED_DIM), jnp.float32)
        for i, cb_ref in enumerate((cb0_ref, cb1_ref, cb2_ref)):
            cb = cb_ref[...]
            sq_ref[i:i + 1, :] = _dot_t(ones, cb * cb) * inv_t
        out_ref[...] = jnp.zeros((1, 1), jnp.float32)

    x = x_ref[...]
    h = jnp.maximum(
        jnp.dot(x.astype(jnp.bfloat16), w1_ref[...],
                preferred_element_type=jnp.float32)
        + b1_ref[...], 0.0)
    res = jnp.dot(h, w2_ref[...], preferred_element_type=jnp.float32) + b2_ref[...]

    CH = BB // NCHAIN
    r = [res[j * CH:(j + 1) * CH] for j in range(NCHAIN)]
    q = [jnp.zeros((CH, EMBED_DIM), jnp.float32) for _ in range(NCHAIN)]
    rqs = [jnp.zeros((CH, 1), jnp.float32) for _ in range(NCHAIN)]
    two_inv_t = 2.0 * inv_t
    for i, cb_ref in enumerate((cb0_ref, cb1_ref, cb2_ref)):
        cb = cb_ref[...]
        sq = sq_ref[i:i + 1, :]
        lg = [_dot_t(r[j] * two_inv_t, cb) - sq for j in range(NCHAIN)]
        for j in range(NCHAIN):
            m = jnp.max(lg[j], axis=1, keepdims=True)
            e = jnp.exp(lg[j] - m)
            d = jnp.sum(e, axis=1, keepdims=True)
            emb = jnp.dot(e, cb, preferred_element_type=jnp.float32) / d
            r[j] = r[j] - emb
            q[j] = q[j] + emb
            rqs[j] = rqs[j] + COMMIT * jnp.sum(r[j] * r[j], axis=1,
                                               keepdims=True)

    quant = jnp.concatenate(q, axis=0)
    rq = jnp.concatenate(rqs, axis=0)
    hd = jnp.maximum(
        jnp.dot(quant, dw1_ref[...], preferred_element_type=jnp.float32)
        + db1_ref[...], 0.0)
    x_hat = jnp.dot(hd.astype(jnp.bfloat16), dw2_ref[...],
                    preferred_element_type=jnp.float32) + db2_ref[...]
    diff = x_hat - x
    recon = jnp.sum(diff * diff, axis=1, keepdims=True)
    out_ref[...] += jnp.sum(recon + rq).reshape(1, 1) / B


def kernel(x, gumbel_t, enc_W1, enc_b1, enc_W2, enc_b2,
           dec_W1, dec_b1, dec_W2, dec_b2, cb0, cb1, cb2):
    t = jnp.asarray(gumbel_t, jnp.float32).reshape(1)
    b1 = enc_b1.reshape(1, HIDDEN_DIM)
    b2 = enc_b2.reshape(1, EMBED_DIM)
    db1 = dec_b1.reshape(1, HIDDEN_DIM)
    db2 = dec_b2.reshape(1, INPUT_DIM)

    const = lambda i: (0, 0)
    out = pl.pallas_call(
        _body,
        grid=(B // BB,),
        in_specs=[
            pl.BlockSpec((BB, INPUT_DIM), lambda i: (i, 0)),
            pl.BlockSpec(memory_space=pltpu.SMEM),
            pl.BlockSpec((INPUT_DIM, HIDDEN_DIM), const),
            pl.BlockSpec((1, HIDDEN_DIM), const),
            pl.BlockSpec((HIDDEN_DIM, EMBED_DIM), const),
            pl.BlockSpec((1, EMBED_DIM), const),
            pl.BlockSpec((EMBED_DIM, HIDDEN_DIM), const),
            pl.BlockSpec((1, HIDDEN_DIM), const),
            pl.BlockSpec((HIDDEN_DIM, INPUT_DIM), const),
            pl.BlockSpec((1, INPUT_DIM), const),
            pl.BlockSpec((K, EMBED_DIM), const),
            pl.BlockSpec((K, EMBED_DIM), const),
            pl.BlockSpec((K, EMBED_DIM), const),
        ],
        out_specs=pl.BlockSpec((1, 1), const),
        out_shape=jax.ShapeDtypeStruct((1, 1), jnp.float32),
        scratch_shapes=[pltpu.VMEM((8, K), jnp.float32)],
        compiler_params=pltpu.CompilerParams(
            dimension_semantics=("arbitrary",)),
    )(x, t, enc_W1.astype(jnp.bfloat16), b1, enc_W2, b2, dec_W1, db1,
      dec_W2.astype(jnp.bfloat16), db2, cb0, cb1, cb2)
    return out[0, 0]


# speedup vs baseline: 1.8911x; 1.0572x over previous
import jax
import jax.numpy as jnp
from jax.experimental import pallas as pl
from jax.experimental.pallas import tpu as pltpu

B, INPUT_DIM, HIDDEN_DIM, EMBED_DIM, K = 2048, 768, 2048, 256, 8192
BB = 512
NCHAIN = 2
COMMIT = 1.25


def _dot_t(a, b):
    return jax.lax.dot_general(a, b, (((1,), (1,)), ((), ())),
                               preferred_element_type=jnp.float32)


def _body(x_ref, t_ref, w1_ref, b1_ref, w2_ref, b2_ref,
          dw1_ref, db1_ref, dw2_ref, db2_ref,
          cb0_ref, cb1_ref, cb2_ref, out_ref, sq_ref):
    inv_t = 1.0 / t_ref[0]

    @pl.when(pl.program_id(0) == 0)
    def _init():
        ones = jnp.ones((1, EMBED_DIM), jnp.float32)
        for i, cb_ref in enumerate((cb0_ref, cb1_ref, cb2_ref)):
            cb = cb_ref[...]
            sq_ref[i:i + 1, :] = _dot_t(ones, cb * cb) * inv_t
        out_ref[...] = jnp.zeros((1, 1), jnp.float32)

    x = x_ref[...]
    h = jnp.maximum(
        jnp.dot(x.astype(jnp.bfloat16), w1_ref[...],
                preferred_element_type=jnp.float32)
        + b1_ref[...], 0.0)
    res = jnp.dot(h, w2_ref[...], preferred_element_type=jnp.float32) + b2_ref[...]

    CH = BB // NCHAIN
    r = [res[j * CH:(j + 1) * CH] for j in range(NCHAIN)]
    q = [jnp.zeros((CH, EMBED_DIM), jnp.float32) for _ in range(NCHAIN)]
    rqs = [jnp.zeros((CH, 1), jnp.float32) for _ in range(NCHAIN)]
    two_inv_t = 2.0 * inv_t
    for i, cb_ref in enumerate((cb0_ref, cb1_ref, cb2_ref)):
        cb = cb_ref[...]
        sq = sq_ref[i:i + 1, :]
        raw = [_dot_t(r[j] * two_inv_t, cb) for j in range(NCHAIN)]
        for j in range(NCHAIN):
            m = jnp.max(raw[j] - sq, axis=1, keepdims=True)
            e = jnp.exp(raw[j] - (sq + m))
            d = jnp.sum(e, axis=1, keepdims=True)
            emb = jnp.dot(e, cb, preferred_element_type=jnp.float32) / d
            r[j] = r[j] - emb
            q[j] = q[j] + emb
            rqs[j] = rqs[j] + COMMIT * jnp.sum(r[j] * r[j], axis=1,
                                               keepdims=True)

    quant = jnp.concatenate(q, axis=0)
    rq = jnp.concatenate(rqs, axis=0)
    hd = jnp.maximum(
        jnp.dot(quant, dw1_ref[...], preferred_element_type=jnp.float32)
        + db1_ref[...], 0.0)
    x_hat = jnp.dot(hd.astype(jnp.bfloat16), dw2_ref[...],
                    preferred_element_type=jnp.float32) + db2_ref[...]
    diff = x_hat - x
    recon = jnp.sum(diff * diff, axis=1, keepdims=True)
    out_ref[...] += jnp.sum(recon + rq).reshape(1, 1) / B


def kernel(x, gumbel_t, enc_W1, enc_b1, enc_W2, enc_b2,
           dec_W1, dec_b1, dec_W2, dec_b2, cb0, cb1, cb2):
    t = jnp.asarray(gumbel_t, jnp.float32).reshape(1)
    b1 = enc_b1.reshape(1, HIDDEN_DIM)
    b2 = enc_b2.reshape(1, EMBED_DIM)
    db1 = dec_b1.reshape(1, HIDDEN_DIM)
    db2 = dec_b2.reshape(1, INPUT_DIM)

    const = lambda i: (0, 0)
    out = pl.pallas_call(
        _body,
        grid=(B // BB,),
        in_specs=[
            pl.BlockSpec((BB, INPUT_DIM), lambda i: (i, 0)),
            pl.BlockSpec(memory_space=pltpu.SMEM),
            pl.BlockSpec((INPUT_DIM, HIDDEN_DIM), const),
            pl.BlockSpec((1, HIDDEN_DIM), const),
            pl.BlockSpec((HIDDEN_DIM, EMBED_DIM), const),
            pl.BlockSpec((1, EMBED_DIM), const),
            pl.BlockSpec((EMBED_DIM, HIDDEN_DIM), const),
            pl.BlockSpec((1, HIDDEN_DIM), const),
            pl.BlockSpec((HIDDEN_DIM, INPUT_DIM), const),
            pl.BlockSpec((1, INPUT_DIM), const),
            pl.BlockSpec((K, EMBED_DIM), const),
            pl.BlockSpec((K, EMBED_DIM), const),
            pl.BlockSpec((K, EMBED_DIM), const),
        ],
        out_specs=pl.BlockSpec((1, 1), const),
        out_shape=jax.ShapeDtypeStruct((1, 1), jnp.float32),
        scratch_shapes=[pltpu.VMEM((8, K), jnp.float32)],
        compiler_params=pltpu.CompilerParams(
            dimension_semantics=("arbitrary",)),
    )(x, t, enc_W1.astype(jnp.bfloat16), b1, enc_W2, b2, dec_W1, db1,
      dec_W2.astype(jnp.bfloat16), db2, cb0, cb1, cb2)
    return out[0, 0]
